# Initial kernel scaffold; baseline (speedup 1.0000x reference)
import jax
import jax.numpy as jnp
from jax.experimental import pallas as pl


def kernel(x, W_h, b_h, W_gcn, b_gcn, W_lin, b_lin, bn_g, bn_b, ln_g, ln_b, W_pred, b_pred, edge_index, walk_node_index, walk_edge_index, walk_pe, num_nodes):
    raise NotImplementedError("write your pallas kernel here")


# speedup vs baseline: 14.8714x; 14.8714x over previous
import functools

import jax
import jax.numpy as jnp
from jax import lax
from jax.experimental import pallas as pl
from jax.experimental.pallas import tpu as pltpu
from jax.experimental.pallas import tpu_sc as plsc

N = 10000
E = 320000
D = 128
OUT = 128
BETA = 0.9
EPS_BN = 1e-5
EPS_LN = 1e-5

NPAD = 10240
NW = 32
ROWS_E = 2560
E_PAD = ROWS_E * 128
RPT = ROWS_E // NW
SLICE = NPAD // 16

_mesh = plsc.VectorSubcoreMesh(core_axis_name="c", subcore_axis_name="s")


@functools.partial(
    pl.kernel,
    mesh=_mesh,
    out_type=jax.ShapeDtypeStruct((2 * NPAD,), jnp.float32),
    scratch_types=[
        pltpu.VMEM((RPT, 128), jnp.int32),
        pltpu.VMEM((128,), jnp.float32),
        pltpu.VMEM((SLICE,), jnp.float32),
        pltpu.VMEM_SHARED((NPAD,), jnp.float32),
    ],
)
def _deg_call(dst_hbm, out_hbm, idx_v, ones_v, z_v, deg_s):
    c = lax.axis_index("c")
    s = lax.axis_index("s")
    wid = c * 16 + s

    def fill(i, _):
        z_v[pl.ds(i * 16, 16)] = jnp.zeros((16,), jnp.float32)
        return 0
    lax.fori_loop(0, SLICE // 16, fill, 0)

    def fill1(i, _):
        ones_v[pl.ds(i * 16, 16)] = jnp.ones((16,), jnp.float32)
        return 0
    lax.fori_loop(0, 8, fill1, 0)

    pltpu.sync_copy(z_v, deg_s.at[pl.ds(s * SLICE, SLICE)])
    plsc.subcore_barrier()

    pltpu.sync_copy(dst_hbm.at[pl.ds(wid * RPT, RPT)], idx_v)

    def body(j, _):
        pltpu.sync_copy(ones_v, deg_s.at[idx_v.at[j]], add=True)
        return 0
    lax.fori_loop(0, RPT, body, 0)

    plsc.subcore_barrier()
    pltpu.sync_copy(deg_s.at[pl.ds(s * SLICE, SLICE)],
                    out_hbm.at[pl.ds(c * NPAD + s * SLICE, SLICE)])


ZROWS = 64


@functools.partial(
    pl.kernel,
    mesh=_mesh,
    out_type=jax.ShapeDtypeStruct((2, NPAD, D), jnp.float32),
    scratch_types=[
        pltpu.VMEM((RPT, 128), jnp.int32),
        pltpu.VMEM((RPT, 128), jnp.int32),
        pltpu.VMEM((128, D), jnp.float32),
        pltpu.VMEM((ZROWS, D), jnp.float32),
        pltpu.VMEM_SHARED((NPAD, D), jnp.float32),
        pltpu.SemaphoreType.DMA,
    ],
)
def _scatter_call(src_hbm, dst_hbm, hgs_hbm, out_hbm,
                  si_v, di_v, rows_v, z_v, acc_s, sem):
    c = lax.axis_index("c")
    s = lax.axis_index("s")
    wid = c * 16 + s

    def fill(i, _):
        z_v[i // 8, pl.ds((i % 8) * 16, 16)] = jnp.zeros((16,), jnp.float32)
        return 0
    lax.fori_loop(0, ZROWS * (D // 16), fill, 0)

    def zero(i, _):
        pltpu.sync_copy(z_v, acc_s.at[pl.ds(s * SLICE + i * ZROWS, ZROWS)])
        return 0
    lax.fori_loop(0, SLICE // ZROWS, zero, 0)
    plsc.subcore_barrier()

    pltpu.sync_copy(src_hbm.at[pl.ds(wid * RPT, RPT)], si_v)
    pltpu.sync_copy(dst_hbm.at[pl.ds(wid * RPT, RPT)], di_v)

    def body(j, _):
        pltpu.async_copy(hgs_hbm.at[si_v.at[j]], rows_v, sem).wait()
        pltpu.sync_copy(rows_v, acc_s.at[di_v.at[j]], add=True)
        return 0
    lax.fori_loop(0, RPT, body, 0)

    plsc.subcore_barrier()
    pltpu.sync_copy(acc_s.at[pl.ds(s * SLICE, SLICE)],
                    out_hbm.at[c, pl.ds(s * SLICE, SLICE)])


BLK = 1000


def _dense1_body(x_ref, wg_ref, wh_ref, bh_ref, wl_ref, bl_ref, cnt_ref,
                 hgs_ref, h_ref, xlin_ref, dis_ref):
    xb = x_ref[...]
    cnt = cnt_ref[...]
    deg = cnt[:, 0:1] + cnt[:, 1:2] + 1.0
    dis = lax.rsqrt(deg)
    hg = jnp.dot(xb, wg_ref[...], preferred_element_type=jnp.float32)
    hgs_ref[...] = hg * dis
    h_ref[...] = jnp.maximum(
        jnp.dot(xb, wh_ref[...], preferred_element_type=jnp.float32)
        + bh_ref[...], 0.0)
    xlin_ref[...] = (
        jnp.dot(xb, wl_ref[...], preferred_element_type=jnp.float32)
        + bl_ref[...])
    dis_ref[...] = dis


_dense1 = pl.pallas_call(
    _dense1_body,
    grid=(N // BLK,),
    in_specs=[
        pl.BlockSpec((BLK, D), lambda i: (i, 0)),
        pl.BlockSpec((D, D), lambda i: (0, 0)),
        pl.BlockSpec((D, D), lambda i: (0, 0)),
        pl.BlockSpec((1, D), lambda i: (0, 0)),
        pl.BlockSpec((D, D), lambda i: (0, 0)),
        pl.BlockSpec((1, D), lambda i: (0, 0)),
        pl.BlockSpec((BLK, 2), lambda i: (i, 0)),
    ],
    out_specs=[
        pl.BlockSpec((BLK, D), lambda i: (i, 0)),
        pl.BlockSpec((BLK, D), lambda i: (i, 0)),
        pl.BlockSpec((BLK, D), lambda i: (i, 0)),
        pl.BlockSpec((BLK, 1), lambda i: (i, 0)),
    ],
    out_shape=[
        jax.ShapeDtypeStruct((N, D), jnp.float32),
        jax.ShapeDtypeStruct((N, D), jnp.float32),
        jax.ShapeDtypeStruct((N, D), jnp.float32),
        jax.ShapeDtypeStruct((N, 1), jnp.float32),
    ],
)


_BN_SCL = (1.0 + EPS_BN) ** -0.5


def _dense2_body(acc_ref, hgs_ref, xlin_ref, h_ref, dis_ref, bgcn_ref,
                 bng_ref, bnb_ref, lng_ref, lnb_ref, wp_ref, bp_ref, out_ref):
    acc = acc_ref[0] + acc_ref[1]
    dis = dis_ref[...]
    gcn = dis * (acc + hgs_ref[...]) + bgcn_ref[...]
    xl = gcn + xlin_ref[...]
    xl = xl * (bng_ref[...] * _BN_SCL) + bnb_ref[...]
    xl = jnp.maximum(xl, 0.0)
    z = h_ref[...] * xl
    mu = jnp.mean(z, axis=-1, keepdims=True)
    var = jnp.mean((z - mu) ** 2, axis=-1, keepdims=True)
    ln = (z - mu) * lax.rsqrt(var + EPS_LN) * lng_ref[...] + lnb_ref[...]
    xl = (1.0 - BETA) * ln + BETA * xl
    out_ref[...] = (
        jnp.dot(xl, wp_ref[...], preferred_element_type=jnp.float32)
        + bp_ref[...])


_dense2 = pl.pallas_call(
    _dense2_body,
    grid=(N // BLK,),
    in_specs=[
        pl.BlockSpec((2, BLK, D), lambda i: (0, i, 0)),
        pl.BlockSpec((BLK, D), lambda i: (i, 0)),
        pl.BlockSpec((BLK, D), lambda i: (i, 0)),
        pl.BlockSpec((BLK, D), lambda i: (i, 0)),
        pl.BlockSpec((BLK, 1), lambda i: (i, 0)),
        pl.BlockSpec((1, D), lambda i: (0, 0)),
        pl.BlockSpec((1, D), lambda i: (0, 0)),
        pl.BlockSpec((1, D), lambda i: (0, 0)),
        pl.BlockSpec((1, D), lambda i: (0, 0)),
        pl.BlockSpec((1, D), lambda i: (0, 0)),
        pl.BlockSpec((D, OUT), lambda i: (0, 0)),
        pl.BlockSpec((1, OUT), lambda i: (0, 0)),
    ],
    out_specs=pl.BlockSpec((BLK, OUT), lambda i: (i, 0)),
    out_shape=jax.ShapeDtypeStruct((N, OUT), jnp.float32),
)


def kernel(x, W_h, b_h, W_gcn, b_gcn, W_lin, b_lin, bn_g, bn_b, ln_g, ln_b,
           W_pred, b_pred, edge_index, walk_node_index, walk_edge_index,
           walk_pe, num_nodes):
    src = edge_index[0].astype(jnp.int32)
    dst = edge_index[1].astype(jnp.int32)
    src2d = jnp.concatenate(
        [src, jnp.zeros((E_PAD - E,), jnp.int32)]).reshape(ROWS_E, 128)
    dst2d = jnp.concatenate(
        [dst, jnp.full((E_PAD - E,), N, jnp.int32)]).reshape(ROWS_E, 128)

    cnt = _deg_call(dst2d).reshape(2, NPAD)
    cnt2 = jnp.transpose(cnt)

    hgs, h, xlin, dis = _dense1(
        x, W_gcn, W_h, b_h.reshape(1, D), W_lin, b_lin.reshape(1, D), cnt2)

    acc = _scatter_call(src2d, dst2d, hgs)

    return _dense2(
        acc, hgs, xlin, h, dis, b_gcn.reshape(1, D), bn_g.reshape(1, D),
        bn_b.reshape(1, D), ln_g.reshape(1, D), ln_b.reshape(1, D),
        W_pred, b_pred.reshape(1, OUT))


# speedup vs baseline: 17.3196x; 1.1646x over previous
import functools

import jax
import jax.numpy as jnp
from jax import lax
from jax.experimental import pallas as pl
from jax.experimental.pallas import tpu as pltpu
from jax.experimental.pallas import tpu_sc as plsc

N = 10000
E = 320000
D = 128
OUT = 128
BETA = 0.9
EPS_BN = 1e-5
EPS_LN = 1e-5

NPAD = 10240
NW = 32
ROWS_E = 2560
E_PAD = ROWS_E * 128
RPT = ROWS_E // NW
SLICE = NPAD // 16

_mesh = plsc.VectorSubcoreMesh(core_axis_name="c", subcore_axis_name="s")


@functools.partial(
    pl.kernel,
    mesh=_mesh,
    out_type=jax.ShapeDtypeStruct((2 * NPAD,), jnp.float32),
    scratch_types=[
        pltpu.VMEM((RPT, 128), jnp.int32),
        pltpu.VMEM((128,), jnp.float32),
        pltpu.VMEM((SLICE,), jnp.float32),
        pltpu.VMEM_SHARED((NPAD,), jnp.float32),
    ],
)
def _deg_call(dst_hbm, out_hbm, idx_v, ones_v, z_v, deg_s):
    c = lax.axis_index("c")
    s = lax.axis_index("s")
    wid = c * 16 + s

    def fill(i, _):
        z_v[pl.ds(i * 16, 16)] = jnp.zeros((16,), jnp.float32)
        return 0
    lax.fori_loop(0, SLICE // 16, fill, 0)

    def fill1(i, _):
        ones_v[pl.ds(i * 16, 16)] = jnp.ones((16,), jnp.float32)
        return 0
    lax.fori_loop(0, 8, fill1, 0)

    pltpu.sync_copy(z_v, deg_s.at[pl.ds(s * SLICE, SLICE)])
    plsc.subcore_barrier()

    pltpu.sync_copy(dst_hbm.at[pl.ds(wid * RPT, RPT)], idx_v)

    def body(j, _):
        pltpu.sync_copy(ones_v, deg_s.at[idx_v.at[j]], add=True)
        return 0
    lax.fori_loop(0, RPT, body, 0)

    plsc.subcore_barrier()
    pltpu.sync_copy(deg_s.at[pl.ds(s * SLICE, SLICE)],
                    out_hbm.at[pl.ds(c * NPAD + s * SLICE, SLICE)])


ZROWS = 16
NBUF = 2
IDXB = RPT // 2


@functools.partial(
    pl.kernel,
    mesh=_mesh,
    out_type=jax.ShapeDtypeStruct((2, NPAD, D), jnp.float32),
    scratch_types=[
        pltpu.VMEM((IDXB, 128), jnp.int32),
        pltpu.VMEM((IDXB, 128), jnp.int32),
        pltpu.VMEM((NBUF, 128, D), jnp.float32),
        pltpu.VMEM((ZROWS, D), jnp.float32),
        pltpu.VMEM_SHARED((NPAD, D), jnp.float32),
        pltpu.SemaphoreType.DMA((NBUF,)),
    ],
)
def _scatter_call(src_hbm, dst_hbm, hgs_hbm, out_hbm,
                  si_v, di_v, rows_v, z_v, acc_s, gsem):
    c = lax.axis_index("c")
    s = lax.axis_index("s")
    wid = c * 16 + s

    def fill(i, _):
        z_v[i // 8, pl.ds((i % 8) * 16, 16)] = jnp.zeros((16,), jnp.float32)
        return 0
    lax.fori_loop(0, ZROWS * (D // 16), fill, 0)

    def zero(i, _):
        pltpu.sync_copy(z_v, acc_s.at[pl.ds(s * SLICE + i * ZROWS, ZROWS)])
        return 0
    lax.fori_loop(0, SLICE // ZROWS, zero, 0)
    plsc.subcore_barrier()

    def _gather(j, b):
        pltpu.async_copy(hgs_hbm.at[si_v.at[j]], rows_v.at[b], gsem.at[b])

    def _wait_g(b):
        pltpu.make_async_copy(hgs_hbm.at[si_v.at[0]], rows_v.at[b],
                              gsem.at[b]).wait()

    def block(p, _):
        base = wid * RPT + p * IDXB
        pltpu.sync_copy(src_hbm.at[pl.ds(base, IDXB)], si_v)
        pltpu.sync_copy(dst_hbm.at[pl.ds(base, IDXB)], di_v)
        _gather(0, 0)

        def body(g, _):
            for k in range(NBUF):
                j = g * NBUF + k

                @pl.when(j + 1 < IDXB)
                def _():
                    _gather(j + 1, (k + 1) % NBUF)

                _wait_g(k)
                pltpu.sync_copy(rows_v.at[k], acc_s.at[di_v.at[j]], add=True)
            return 0
        lax.fori_loop(0, IDXB // NBUF, body, 0)
        return 0
    lax.fori_loop(0, 2, block, 0)

    plsc.subcore_barrier()
    pltpu.sync_copy(acc_s.at[pl.ds(s * SLICE, SLICE)],
                    out_hbm.at[c, pl.ds(s * SLICE, SLICE)])


BLK = 1000


def _dense1_body(x_ref, wg_ref, wh_ref, bh_ref, wl_ref, bl_ref, cnt_ref,
                 hgs_ref, h_ref, xlin_ref, dis_ref):
    xb = x_ref[...]
    cnt = cnt_ref[...]
    deg = cnt[:, 0:1] + cnt[:, 1:2] + 1.0
    dis = lax.rsqrt(deg)
    hg = jnp.dot(xb, wg_ref[...], preferred_element_type=jnp.float32)
    hgs_ref[...] = hg * dis
    h_ref[...] = jnp.maximum(
        jnp.dot(xb, wh_ref[...], preferred_element_type=jnp.float32)
        + bh_ref[...], 0.0)
    xlin_ref[...] = (
        jnp.dot(xb, wl_ref[...], preferred_element_type=jnp.float32)
        + bl_ref[...])
    dis_ref[...] = dis


_dense1 = pl.pallas_call(
    _dense1_body,
    grid=(N // BLK,),
    in_specs=[
        pl.BlockSpec((BLK, D), lambda i: (i, 0)),
        pl.BlockSpec((D, D), lambda i: (0, 0)),
        pl.BlockSpec((D, D), lambda i: (0, 0)),
        pl.BlockSpec((1, D), lambda i: (0, 0)),
        pl.BlockSpec((D, D), lambda i: (0, 0)),
        pl.BlockSpec((1, D), lambda i: (0, 0)),
        pl.BlockSpec((BLK, 2), lambda i: (i, 0)),
    ],
    out_specs=[
        pl.BlockSpec((BLK, D), lambda i: (i, 0)),
        pl.BlockSpec((BLK, D), lambda i: (i, 0)),
        pl.BlockSpec((BLK, D), lambda i: (i, 0)),
        pl.BlockSpec((BLK, 1), lambda i: (i, 0)),
    ],
    out_shape=[
        jax.ShapeDtypeStruct((N, D), jnp.float32),
        jax.ShapeDtypeStruct((N, D), jnp.float32),
        jax.ShapeDtypeStruct((N, D), jnp.float32),
        jax.ShapeDtypeStruct((N, 1), jnp.float32),
    ],
)


_BN_SCL = (1.0 + EPS_BN) ** -0.5


def _dense2_body(acc_ref, hgs_ref, xlin_ref, h_ref, dis_ref, bgcn_ref,
                 bng_ref, bnb_ref, lng_ref, lnb_ref, wp_ref, bp_ref, out_ref):
    acc = acc_ref[0] + acc_ref[1]
    dis = dis_ref[...]
    gcn = dis * (acc + hgs_ref[...]) + bgcn_ref[...]
    xl = gcn + xlin_ref[...]
    xl = xl * (bng_ref[...] * _BN_SCL) + bnb_ref[...]
    xl = jnp.maximum(xl, 0.0)
    z = h_ref[...] * xl
    mu = jnp.mean(z, axis=-1, keepdims=True)
    var = jnp.mean((z - mu) ** 2, axis=-1, keepdims=True)
    ln = (z - mu) * lax.rsqrt(var + EPS_LN) * lng_ref[...] + lnb_ref[...]
    xl = (1.0 - BETA) * ln + BETA * xl
    out_ref[...] = (
        jnp.dot(xl, wp_ref[...], preferred_element_type=jnp.float32)
        + bp_ref[...])


_dense2 = pl.pallas_call(
    _dense2_body,
    grid=(N // BLK,),
    in_specs=[
        pl.BlockSpec((2, BLK, D), lambda i: (0, i, 0)),
        pl.BlockSpec((BLK, D), lambda i: (i, 0)),
        pl.BlockSpec((BLK, D), lambda i: (i, 0)),
        pl.BlockSpec((BLK, D), lambda i: (i, 0)),
        pl.BlockSpec((BLK, 1), lambda i: (i, 0)),
        pl.BlockSpec((1, D), lambda i: (0, 0)),
        pl.BlockSpec((1, D), lambda i: (0, 0)),
        pl.BlockSpec((1, D), lambda i: (0, 0)),
        pl.BlockSpec((1, D), lambda i: (0, 0)),
        pl.BlockSpec((1, D), lambda i: (0, 0)),
        pl.BlockSpec((D, OUT), lambda i: (0, 0)),
        pl.BlockSpec((1, OUT), lambda i: (0, 0)),
    ],
    out_specs=pl.BlockSpec((BLK, OUT), lambda i: (i, 0)),
    out_shape=jax.ShapeDtypeStruct((N, OUT), jnp.float32),
)


def kernel(x, W_h, b_h, W_gcn, b_gcn, W_lin, b_lin, bn_g, bn_b, ln_g, ln_b,
           W_pred, b_pred, edge_index, walk_node_index, walk_edge_index,
           walk_pe, num_nodes):
    src = edge_index[0].astype(jnp.int32)
    dst = edge_index[1].astype(jnp.int32)
    src2d = jnp.concatenate(
        [src, jnp.zeros((E_PAD - E,), jnp.int32)]).reshape(ROWS_E, 128)
    dst2d = jnp.concatenate(
        [dst, jnp.full((E_PAD - E,), N, jnp.int32)]).reshape(ROWS_E, 128)

    cnt = _deg_call(dst2d).reshape(2, NPAD)
    cnt2 = jnp.transpose(cnt)

    hgs, h, xlin, dis = _dense1(
        x, W_gcn, W_h, b_h.reshape(1, D), W_lin, b_lin.reshape(1, D), cnt2)

    acc = _scatter_call(src2d, dst2d, hgs)

    return _dense2(
        acc, hgs, xlin, h, dis, b_gcn.reshape(1, D), bn_g.reshape(1, D),
        bn_b.reshape(1, D), ln_g.reshape(1, D), ln_b.reshape(1, D),
        W_pred, b_pred.reshape(1, OUT))
